# baseline (speedup 1.0000x reference)
import functools

import jax
import jax.numpy as jnp
from jax import lax
from jax.experimental import pallas as pl
from jax.experimental.pallas import tpu as pltpu
from jax.experimental.pallas import tpu_sc as plsc

_D = 32
_NE = 8192
_NTOK = 8192
_BLK_T = 256
_CB_CHUNK = 512
_FOLD = 2048
_CPF = _FOLD // _CB_CHUNK


def _tc_body(xt_ref, emb_ref, xsq_ref, esq_ref, idx_ref, dsum_ref,
             blkd_ref, blki_ref, accv_ref, acci_ref, accd_ref):
    j = pl.program_id(1)
    xt = xt_ref[...]
    e = emb_ref[...]
    xsq = xsq_ref[...]
    esq = esq_ref[...]
    prod = lax.dot_general(
        e, xt, (((1,), (0,)), ((), ())),
        preferred_element_type=jnp.float32,
    )
    scores = (xsq + esq) - 2.0 * prod
    dmin = jnp.min(scores, axis=0, keepdims=True)
    iot = lax.broadcasted_iota(jnp.int32, scores.shape, 0)
    masked = jnp.where(scores <= dmin, iot, _NE)
    imin = jnp.min(masked, axis=0, keepdims=True) + j * _CB_CHUNK

    c = j % _CPF

    @pl.when(c == 0)
    def _():
        blkd_ref[...] = dmin
        blki_ref[...] = imin

    @pl.when(c > 0)
    def _():
        prev_d = blkd_ref[...]
        upd = dmin < prev_d
        blkd_ref[...] = jnp.where(upd, dmin, prev_d)
        blki_ref[...] = jnp.where(upd, imin, blki_ref[...])

    @pl.when(j == 0)
    def _():
        accv_ref[...] = jnp.full_like(accv_ref, jnp.inf)
        acci_ref[...] = jnp.zeros_like(acci_ref)
        accd_ref[...] = jnp.zeros_like(accd_ref)

    @pl.when((pl.program_id(0) == 0) & (j == 0))
    def _():
        dsum_ref[...] = jnp.zeros_like(dsum_ref)

    @pl.when(c == _CPF - 1)
    def _():
        bd = blkd_ref[...]
        lt = bd < accv_ref[...]
        accv_ref[...] = jnp.where(
            lt, bd.astype(jnp.bfloat16).astype(jnp.float32), accv_ref[...])
        acci_ref[...] = jnp.where(lt, blki_ref[...], acci_ref[...])
        accd_ref[...] = jnp.where(lt, bd, accd_ref[...])

    @pl.when(j == pl.num_programs(1) - 1)
    def _():
        idx_ref[0, :, :] = acci_ref[...]
        dsum_ref[...] += jnp.sum(accd_ref[...]).reshape(1, 1)


def _tc_argmin(xt, emb, xsq, esq):
    nblk = _NTOK // _BLK_T
    ncb = _NE // _CB_CHUNK
    idx3, dsum = pl.pallas_call(
        _tc_body,
        grid=(nblk, ncb),
        in_specs=[
            pl.BlockSpec((_D, _BLK_T), lambda i, j: (0, i)),
            pl.BlockSpec((_CB_CHUNK, _D), lambda i, j: (j, 0)),
            pl.BlockSpec((1, _BLK_T), lambda i, j: (0, i)),
            pl.BlockSpec((_CB_CHUNK, 1), lambda i, j: (j, 0)),
        ],
        out_specs=[
            pl.BlockSpec((1, 1, _BLK_T), lambda i, j: (i, 0, 0)),
            pl.BlockSpec((1, 1), lambda i, j: (0, 0)),
        ],
        out_shape=[
            jax.ShapeDtypeStruct((nblk, 1, _BLK_T), jnp.int32),
            jax.ShapeDtypeStruct((1, 1), jnp.float32),
        ],
        scratch_shapes=[
            pltpu.VMEM((1, _BLK_T), jnp.float32),
            pltpu.VMEM((1, _BLK_T), jnp.int32),
            pltpu.VMEM((1, _BLK_T), jnp.float32),
            pltpu.VMEM((1, _BLK_T), jnp.int32),
            pltpu.VMEM((1, _BLK_T), jnp.float32),
        ],
    )(xt, emb, xsq, esq)
    return idx3.reshape(-1), dsum[0, 0]


def _sc_gather(table, idx):
    info = plsc.get_sparse_core_info()
    nc, ns = info.num_cores, info.num_subcores
    nw = nc * ns
    b_per_w = _NTOK // nw
    mesh = plsc.VectorSubcoreMesh(core_axis_name="c", subcore_axis_name="s")

    @functools.partial(
        pl.kernel,
        mesh=mesh,
        compiler_params=pltpu.CompilerParams(use_tc_tiling_on_sc=False),
        out_type=jax.ShapeDtypeStruct((_NTOK, _D), jnp.float32),
        scratch_types=[
            pltpu.VMEM((b_per_w,), jnp.int32),
            pltpu.VMEM((b_per_w, _D), jnp.float32),
            pltpu.SemaphoreType.DMA,
        ],
    )
    def k(table_hbm, idx_hbm, out_hbm, idx_v, rows_v, sem):
        wid = lax.axis_index("s") * nc + lax.axis_index("c")
        base = wid * b_per_w
        pltpu.sync_copy(idx_hbm.at[pl.ds(base, b_per_w)], idx_v)
        pltpu.async_copy(table_hbm.at[idx_v], rows_v, sem).wait()
        pltpu.sync_copy(rows_v, out_hbm.at[pl.ds(base, b_per_w)])

    return k(table, idx)


def kernel(x, embedding_weight):
    flat = x.reshape(-1, _D)
    xsq = jnp.sum(flat ** 2, axis=1).reshape(1, _NTOK)
    esq = jnp.sum(embedding_weight ** 2, axis=1).reshape(_NE, 1)
    bmu_idx, dsum = _tc_argmin(flat.T, embedding_weight, xsq, esq)
    quantized = _sc_gather(embedding_weight, bmu_idx).reshape(x.shape)
    diff = dsum / jnp.float32(x.size)
    return quantized, diff


# speedup vs baseline: 3.3201x; 3.3201x over previous
import functools

import jax
import jax.numpy as jnp
from jax import lax
from jax.experimental import pallas as pl
from jax.experimental.pallas import tpu as pltpu
from jax.experimental.pallas import tpu_sc as plsc

_D = 32
_NE = 8192
_NTOK = 8192
_BLK_T = 8192
_CB_CHUNK = 512
_FOLD = 2048
_CPF = _FOLD // _CB_CHUNK


def _tc_body(xt_ref, emb_ref, xsq_ref, esq_ref, idx_ref, dsum_ref,
             blkd_ref, blki_ref, accv_ref, acci_ref, accd_ref):
    j = pl.program_id(1)
    xt = xt_ref[...]
    e = emb_ref[...]
    xsq = xsq_ref[...]
    esq = esq_ref[...]
    prod = lax.dot_general(
        e, xt, (((1,), (0,)), ((), ())),
        preferred_element_type=jnp.float32,
    )
    scores = (xsq + esq) - 2.0 * prod
    dmin = jnp.min(scores, axis=0, keepdims=True)
    iot = lax.broadcasted_iota(jnp.int32, scores.shape, 0)
    masked = jnp.where(scores <= dmin, iot, _NE)
    imin = jnp.min(masked, axis=0, keepdims=True) + j * _CB_CHUNK

    c = j % _CPF

    @pl.when(c == 0)
    def _():
        blkd_ref[...] = dmin
        blki_ref[...] = imin

    @pl.when(c > 0)
    def _():
        prev_d = blkd_ref[...]
        upd = dmin < prev_d
        blkd_ref[...] = jnp.where(upd, dmin, prev_d)
        blki_ref[...] = jnp.where(upd, imin, blki_ref[...])

    @pl.when(j == 0)
    def _():
        accv_ref[...] = jnp.full_like(accv_ref, jnp.inf)
        acci_ref[...] = jnp.zeros_like(acci_ref)
        accd_ref[...] = jnp.zeros_like(accd_ref)

    @pl.when((pl.program_id(0) == 0) & (j == 0))
    def _():
        dsum_ref[...] = jnp.zeros_like(dsum_ref)

    @pl.when(c == _CPF - 1)
    def _():
        bd = blkd_ref[...]
        lt = bd < accv_ref[...]
        accv_ref[...] = jnp.where(
            lt, bd.astype(jnp.bfloat16).astype(jnp.float32), accv_ref[...])
        acci_ref[...] = jnp.where(lt, blki_ref[...], acci_ref[...])
        accd_ref[...] = jnp.where(lt, bd, accd_ref[...])

    @pl.when(j == pl.num_programs(1) - 1)
    def _():
        idx_ref[0, :, :] = acci_ref[...]
        dsum_ref[...] += jnp.sum(accd_ref[...]).reshape(1, 1)


def _tc_argmin(xt, emb, xsq, esq):
    nblk = _NTOK // _BLK_T
    ncb = _NE // _CB_CHUNK
    idx3, dsum = pl.pallas_call(
        _tc_body,
        grid=(nblk, ncb),
        in_specs=[
            pl.BlockSpec((_D, _BLK_T), lambda i, j: (0, i)),
            pl.BlockSpec((_CB_CHUNK, _D), lambda i, j: (j, 0)),
            pl.BlockSpec((1, _BLK_T), lambda i, j: (0, i)),
            pl.BlockSpec((_CB_CHUNK, 1), lambda i, j: (j, 0)),
        ],
        out_specs=[
            pl.BlockSpec((1, 1, _BLK_T), lambda i, j: (i, 0, 0)),
            pl.BlockSpec((1, 1), lambda i, j: (0, 0)),
        ],
        out_shape=[
            jax.ShapeDtypeStruct((nblk, 1, _BLK_T), jnp.int32),
            jax.ShapeDtypeStruct((1, 1), jnp.float32),
        ],
        scratch_shapes=[
            pltpu.VMEM((1, _BLK_T), jnp.float32),
            pltpu.VMEM((1, _BLK_T), jnp.int32),
            pltpu.VMEM((1, _BLK_T), jnp.float32),
            pltpu.VMEM((1, _BLK_T), jnp.int32),
            pltpu.VMEM((1, _BLK_T), jnp.float32),
        ],
    )(xt, emb, xsq, esq)
    return idx3.reshape(-1), dsum[0, 0]


def _sc_gather(table, idx):
    info = plsc.get_sparse_core_info()
    nc, ns = info.num_cores, info.num_subcores
    nw = nc * ns
    b_per_w = _NTOK // nw
    mesh = plsc.VectorSubcoreMesh(core_axis_name="c", subcore_axis_name="s")

    @functools.partial(
        pl.kernel,
        mesh=mesh,
        compiler_params=pltpu.CompilerParams(use_tc_tiling_on_sc=False),
        out_type=jax.ShapeDtypeStruct((_NTOK, _D), jnp.float32),
        scratch_types=[
            pltpu.VMEM((b_per_w,), jnp.int32),
            pltpu.VMEM((b_per_w, _D), jnp.float32),
            pltpu.SemaphoreType.DMA,
        ],
    )
    def k(table_hbm, idx_hbm, out_hbm, idx_v, rows_v, sem):
        wid = lax.axis_index("s") * nc + lax.axis_index("c")
        base = wid * b_per_w
        pltpu.sync_copy(idx_hbm.at[pl.ds(base, b_per_w)], idx_v)
        pltpu.async_copy(table_hbm.at[idx_v], rows_v, sem).wait()
        pltpu.sync_copy(rows_v, out_hbm.at[pl.ds(base, b_per_w)])

    return k(table, idx)


def kernel(x, embedding_weight):
    flat = x.reshape(-1, _D)
    xsq = jnp.sum(flat ** 2, axis=1).reshape(1, _NTOK)
    esq = jnp.sum(embedding_weight ** 2, axis=1).reshape(_NE, 1)
    bmu_idx, dsum = _tc_argmin(flat.T, embedding_weight, xsq, esq)
    quantized = _sc_gather(embedding_weight, bmu_idx).reshape(x.shape)
    diff = dsum / jnp.float32(x.size)
    return quantized, diff


# speedup vs baseline: 3.7643x; 1.1338x over previous
import functools

import jax
import jax.numpy as jnp
from jax import lax
from jax.experimental import pallas as pl
from jax.experimental.pallas import tpu as pltpu
from jax.experimental.pallas import tpu_sc as plsc

_D = 32
_NE = 8192
_NTOK = 8192
_BLK_T = 8192
_CB_CHUNK = 2048
_FOLD = 2048
_CPF = _FOLD // _CB_CHUNK


def _tc_body(xt_ref, emb_ref, xsq_ref, esq_ref, idx_ref, dsum_ref,
             blkd_ref, blki_ref, accv_ref, acci_ref, accd_ref):
    j = pl.program_id(1)
    xt = xt_ref[...]
    e = emb_ref[...]
    xsq = xsq_ref[...]
    esq = esq_ref[...]
    prod2 = lax.dot_general(
        e, xt, (((1,), (0,)), ((), ())),
        preferred_element_type=jnp.float32,
    )
    scores = (xsq + esq) - prod2
    dmin = jnp.min(scores, axis=0, keepdims=True)
    iot = lax.broadcasted_iota(jnp.int32, scores.shape, 0)
    masked = jnp.where(scores <= dmin, iot, _NE)
    imin = jnp.min(masked, axis=0, keepdims=True) + j * _CB_CHUNK

    c = j % _CPF

    @pl.when(c == 0)
    def _():
        blkd_ref[...] = dmin
        blki_ref[...] = imin

    @pl.when(c > 0)
    def _():
        prev_d = blkd_ref[...]
        upd = dmin < prev_d
        blkd_ref[...] = jnp.where(upd, dmin, prev_d)
        blki_ref[...] = jnp.where(upd, imin, blki_ref[...])

    @pl.when(j == 0)
    def _():
        accv_ref[...] = jnp.full_like(accv_ref, jnp.inf)
        acci_ref[...] = jnp.zeros_like(acci_ref)
        accd_ref[...] = jnp.zeros_like(accd_ref)

    @pl.when((pl.program_id(0) == 0) & (j == 0))
    def _():
        dsum_ref[...] = jnp.zeros_like(dsum_ref)

    @pl.when(c == _CPF - 1)
    def _():
        bd = blkd_ref[...]
        lt = bd < accv_ref[...]
        accv_ref[...] = jnp.where(
            lt, bd.astype(jnp.bfloat16).astype(jnp.float32), accv_ref[...])
        acci_ref[...] = jnp.where(lt, blki_ref[...], acci_ref[...])
        accd_ref[...] = jnp.where(lt, bd, accd_ref[...])

    @pl.when(j == pl.num_programs(1) - 1)
    def _():
        idx_ref[0, :, :] = acci_ref[...]
        dsum_ref[...] += jnp.sum(accd_ref[...]).reshape(1, 1)


def _tc_argmin(xt, emb, xsq, esq):
    nblk = _NTOK // _BLK_T
    ncb = _NE // _CB_CHUNK
    idx3, dsum = pl.pallas_call(
        _tc_body,
        grid=(nblk, ncb),
        in_specs=[
            pl.BlockSpec((_D, _BLK_T), lambda i, j: (0, i)),
            pl.BlockSpec((_CB_CHUNK, _D), lambda i, j: (j, 0)),
            pl.BlockSpec((1, _BLK_T), lambda i, j: (0, i)),
            pl.BlockSpec((_CB_CHUNK, 1), lambda i, j: (j, 0)),
        ],
        out_specs=[
            pl.BlockSpec((1, 1, _BLK_T), lambda i, j: (i, 0, 0)),
            pl.BlockSpec((1, 1), lambda i, j: (0, 0)),
        ],
        out_shape=[
            jax.ShapeDtypeStruct((nblk, 1, _BLK_T), jnp.int32),
            jax.ShapeDtypeStruct((1, 1), jnp.float32),
        ],
        scratch_shapes=[
            pltpu.VMEM((1, _BLK_T), jnp.float32),
            pltpu.VMEM((1, _BLK_T), jnp.int32),
            pltpu.VMEM((1, _BLK_T), jnp.float32),
            pltpu.VMEM((1, _BLK_T), jnp.int32),
            pltpu.VMEM((1, _BLK_T), jnp.float32),
        ],
    )(xt, emb, xsq, esq)
    return idx3.reshape(-1), dsum[0, 0]


def _sc_gather(table, idx):
    info = plsc.get_sparse_core_info()
    nc, ns = info.num_cores, info.num_subcores
    nw = nc * ns
    b_per_w = _NTOK // nw
    mesh = plsc.VectorSubcoreMesh(core_axis_name="c", subcore_axis_name="s")

    @functools.partial(
        pl.kernel,
        mesh=mesh,
        compiler_params=pltpu.CompilerParams(use_tc_tiling_on_sc=False),
        out_type=jax.ShapeDtypeStruct((_NTOK, _D), jnp.float32),
        scratch_types=[
            pltpu.VMEM((b_per_w,), jnp.int32),
            pltpu.VMEM((b_per_w, _D), jnp.float32),
            pltpu.SemaphoreType.DMA,
        ],
    )
    def k(table_hbm, idx_hbm, out_hbm, idx_v, rows_v, sem):
        wid = lax.axis_index("s") * nc + lax.axis_index("c")
        base = wid * b_per_w
        pltpu.sync_copy(idx_hbm.at[pl.ds(base, b_per_w)], idx_v)
        pltpu.async_copy(table_hbm.at[idx_v], rows_v, sem).wait()
        pltpu.sync_copy(rows_v, out_hbm.at[pl.ds(base, b_per_w)])

    return k(table, idx)


def kernel(x, embedding_weight):
    flat = x.reshape(-1, _D)
    xsq = jnp.sum(flat ** 2, axis=1).reshape(1, _NTOK)
    esq = jnp.sum(embedding_weight ** 2, axis=1).reshape(_NE, 1)
    bmu_idx, dsum = _tc_argmin(flat.T * 2.0, embedding_weight, xsq, esq)
    quantized = _sc_gather(embedding_weight, bmu_idx).reshape(x.shape)
    diff = dsum / jnp.float32(x.size)
    return quantized, diff
